# Initial kernel scaffold; baseline (speedup 1.0000x reference)
import jax
import jax.numpy as jnp
from jax.experimental import pallas as pl


def kernel(input, target, cur_epoch):
    raise NotImplementedError("write your pallas kernel here")


# speedup vs baseline: 9.5675x; 9.5675x over previous
import functools

import jax
import jax.numpy as jnp
from jax import lax
from jax.experimental import pallas as pl
from jax.experimental.pallas import tpu as pltpu
from jax.experimental.pallas import tpu_sc as plsc

_START_WARM = 5000
_END_WARM = 15000
_TOP_P = 0.15
_CUR_EPOCH_VALUE = 10000

_ROWS, _COLS = 4096, 1024
_N = _ROWS * _COLS

if _CUR_EPOCH_VALUE > _END_WARM:
    _THIS_P = _TOP_P
else:
    _THIS_P = _TOP_P + (1.0 - _TOP_P) * (
        (_END_WARM - _CUR_EPOCH_VALUE) / (_END_WARM - _START_WARM))
_K = int(_N * _THIS_P)

_NC, _NS, _L = 2, 16, 16
_NW = _NC * _NS
_PER_W = _N // _NW
_CHUNK = 4096
_NCHUNK = _PER_W // _CHUNK

_B1, _B2, _B3 = 2048, 4096, 256


def _bce_body(x_ref, t_ref, raw_ref, psum_ref):
    x = x_ref[...]
    t = t_ref[...]
    raw = jnp.maximum(x, 0.0) - x * t + jnp.log1p(jnp.exp(-jnp.abs(x)))
    raw_ref[...] = raw
    psum_ref[pl.program_id(0)] = jnp.sum(raw)


_BCE_GRID = 8
_BCE_BR = _ROWS // _BCE_GRID


def _bce(x2d, t2d):
    return pl.pallas_call(
        _bce_body,
        grid=(_BCE_GRID,),
        in_specs=[pl.BlockSpec((_BCE_BR, _COLS), lambda i: (i, 0)),
                  pl.BlockSpec((_BCE_BR, _COLS), lambda i: (i, 0))],
        out_specs=[pl.BlockSpec((_BCE_BR, _COLS), lambda i: (i, 0)),
                   pl.BlockSpec(memory_space=pltpu.SMEM)],
        out_shape=[jax.ShapeDtypeStruct((_ROWS, _COLS), jnp.float32),
                   jax.ShapeDtypeStruct((_BCE_GRID,), jnp.float32)],
    )(x2d, t2d)


def _zero_hist(hist_v, words):
    zeros16 = jnp.zeros((_L,), jnp.int32)

    def zbody(i, c):
        for j in range(8):
            hist_v[pl.ds(i * 8 * _L + j * _L, _L)] = zeros16
        return c

    lax.fori_loop(0, words // (8 * _L), zbody, 0)


def _mesh():
    return plsc.VectorSubcoreMesh(
        core_axis_name="c", subcore_axis_name="s", num_cores=_NC)


def _wid():
    return lax.axis_index("c") * _NS + lax.axis_index("s")


def _make_pass1():
    words = _B1 * _L

    @functools.partial(
        pl.kernel,
        mesh=_mesh(),
        out_type=[jax.ShapeDtypeStruct((_NW * words,), jnp.int32)],
        scratch_types=[pltpu.VMEM((words,), jnp.int32),
                       pltpu.VMEM((_CHUNK,), jnp.float32)],
        compiler_params=pltpu.CompilerParams(needs_layout_passes=False),
    )
    def pass1(raw_hbm, hist_out, hist_v, buf_v):
        wid = _wid()
        base = wid * _PER_W
        _zero_hist(hist_v, words)
        lane = lax.iota(jnp.int32, _L)
        ones = jnp.ones((_L,), jnp.int32)

        def cbody(c, carry):
            pltpu.sync_copy(raw_hbm.at[pl.ds(base + c * _CHUNK, _CHUNK)],
                            buf_v)

            def vbody(i, carry2):
                v = buf_v[pl.ds(i * _L, _L)]
                key = plsc.bitcast(v, jnp.int32)
                b1 = jax.lax.shift_right_logical(key, 20)
                idx = b1 * _L + lane
                plsc.addupdate_scatter(hist_v, [idx], ones)
                return carry2

            return lax.fori_loop(0, _CHUNK // _L, vbody, carry)

        lax.fori_loop(0, _NCHUNK, cbody, 0)
        pltpu.sync_copy(hist_v, hist_out.at[pl.ds(wid * words, words)])

    return pass1


def _make_pass23(level):
    nbins = _B2 if level == 2 else _B3
    words = nbins * _L
    sel_len = _L if level == 2 else 2 * _L

    @functools.partial(
        pl.kernel,
        mesh=_mesh(),
        out_type=[jax.ShapeDtypeStruct((_NW * words,), jnp.int32),
                  jax.ShapeDtypeStruct((_NW * _L,), jnp.float32)],
        scratch_types=[pltpu.VMEM((words,), jnp.int32),
                       pltpu.VMEM((_CHUNK,), jnp.float32),
                       pltpu.VMEM((_L,), jnp.float32),
                       pltpu.VMEM((sel_len,), jnp.int32)],
        compiler_params=pltpu.CompilerParams(needs_layout_passes=False),
    )
    def pass23(raw_hbm, sel_hbm, hist_out, acc_out, hist_v, buf_v, acc_v,
               sel_v):
        wid = _wid()
        base = wid * _PER_W
        _zero_hist(hist_v, words)
        pltpu.sync_copy(sel_hbm, sel_v)
        sel_a = sel_v[pl.ds(0, _L)]
        sel_b = sel_v[pl.ds(_L, _L)] if level == 3 else None
        lane = lax.iota(jnp.int32, _L)
        ones = jnp.ones((_L,), jnp.int32)

        def cbody(c, acc):
            pltpu.sync_copy(raw_hbm.at[pl.ds(base + c * _CHUNK, _CHUNK)],
                            buf_v)

            def vbody(i, acc2):
                v = buf_v[pl.ds(i * _L, _L)]
                key = plsc.bitcast(v, jnp.int32)
                if level == 2:
                    f = jax.lax.shift_right_logical(key, 20)
                    m_eq = f == sel_a
                    m_gt = f > sel_a
                    b = jnp.bitwise_and(
                        jax.lax.shift_right_logical(key, 8), 0xFFF)
                else:
                    p2 = jax.lax.shift_right_logical(key, 8)
                    m_eq = p2 == sel_a
                    m_gt = jnp.logical_and(p2 > sel_a, p2 < sel_b)
                    b = jnp.bitwise_and(key, 0xFF)
                idx = b * _L + lane
                plsc.addupdate_scatter(hist_v, [idx], ones, mask=m_eq)
                return acc2 + jnp.where(m_gt, v, 0.0)

            return lax.fori_loop(0, _CHUNK // _L, vbody, acc)

        acc = lax.fori_loop(0, _NCHUNK, cbody, jnp.zeros((_L,), jnp.float32))
        pltpu.sync_copy(hist_v, hist_out.at[pl.ds(wid * words, words)])
        acc_v[...] = acc
        pltpu.sync_copy(acc_v, acc_out.at[pl.ds(wid * _L, _L)])

    return pass23


_pass1 = _make_pass1()
_pass2 = _make_pass23(2)
_pass3 = _make_pass23(3)


def _select(counts, kk):
    c = counts
    above = jnp.cumsum(c[::-1])[::-1] - c
    cond = jnp.logical_and(above < kk, above + c >= kk)
    b = jnp.argmax(cond).astype(jnp.int32)
    return b, above[b]


def kernel(input, target, cur_epoch):
    x = input.reshape(_ROWS, _COLS)
    t = target.reshape(_ROWS, _COLS)
    raw2d, psums = _bce(x, t)
    raw = raw2d.reshape(-1)
    warm = jnp.sum(psums) / _N

    (h1,) = (_pass1(raw),)
    if isinstance(h1, (tuple, list)):
        h1 = h1[0]
    c1 = jnp.sum(h1.reshape(_NW, _B1, _L), axis=(0, 2))
    b1, s1 = _select(c1, _K)
    k1 = _K - s1

    sel2 = jnp.broadcast_to(b1, (_L,)).astype(jnp.int32)
    h2, a2 = _pass2(raw, sel2)
    c2 = jnp.sum(h2.reshape(_NW, _B2, _L), axis=(0, 2))
    sum_hi2 = jnp.sum(a2)
    b2, s2 = _select(c2, k1)
    k2 = k1 - s2

    pref2 = b1 * 4096 + b2
    hi2 = (b1 + 1) * 4096
    sel3 = jnp.concatenate([
        jnp.broadcast_to(pref2, (_L,)),
        jnp.broadcast_to(hi2, (_L,)),
    ]).astype(jnp.int32)
    h3, a3 = _pass3(raw, sel3)
    c3 = jnp.sum(h3.reshape(_NW, _B3, _L), axis=(0, 2))
    sum_hi3 = jnp.sum(a3)
    b3, s3 = _select(c3, k2)

    cnt_gt = s1 + s2 + s3
    base_key = pref2 * 256
    keys3 = base_key + jnp.arange(_B3, dtype=jnp.int32)
    vals3 = lax.bitcast_convert_type(keys3, jnp.float32)
    bins3 = jnp.arange(_B3, dtype=jnp.int32)
    sum_rec = jnp.sum(
        jnp.where(bins3 > b3, c3.astype(jnp.float32) * vals3, 0.0))
    tau = lax.bitcast_convert_type((base_key + b3).astype(jnp.int32),
                                   jnp.float32)
    sum_gt = sum_hi2 + sum_hi3 + sum_rec
    ties = (_K - cnt_gt).astype(jnp.float32)
    mean_top = (sum_gt + ties * tau) / _K
    return jnp.where(cur_epoch < _START_WARM, warm, mean_top)


# speedup vs baseline: 11.5041x; 1.2024x over previous
import functools

import jax
import jax.numpy as jnp
from jax import lax
from jax.experimental import pallas as pl
from jax.experimental.pallas import tpu as pltpu
from jax.experimental.pallas import tpu_sc as plsc

_START_WARM = 5000
_END_WARM = 15000
_TOP_P = 0.15
_CUR_EPOCH_VALUE = 10000

_ROWS, _COLS = 4096, 1024
_N = _ROWS * _COLS

if _CUR_EPOCH_VALUE > _END_WARM:
    _THIS_P = _TOP_P
else:
    _THIS_P = _TOP_P + (1.0 - _TOP_P) * (
        (_END_WARM - _CUR_EPOCH_VALUE) / (_END_WARM - _START_WARM))
_K = int(_N * _THIS_P)

_NC, _NS, _L = 2, 16, 16
_NW = _NC * _NS
_PER_W = _N // _NW
_CHUNK = 8192
_NCHUNK = _PER_W // _CHUNK
_UNROLL = 4

_B1, _B2, _B3 = 2048, 4096, 256


def _bce_body(x_ref, t_ref, raw_ref, psum_ref):
    x = x_ref[...]
    t = t_ref[...]
    raw = jnp.maximum(x, 0.0) - x * t + jnp.log1p(jnp.exp(-jnp.abs(x)))
    raw_ref[...] = raw
    psum_ref[pl.program_id(0)] = jnp.sum(raw)


_BCE_GRID = 8
_BCE_BR = _ROWS // _BCE_GRID


def _bce(x2d, t2d):
    return pl.pallas_call(
        _bce_body,
        grid=(_BCE_GRID,),
        in_specs=[pl.BlockSpec((_BCE_BR, _COLS), lambda i: (i, 0)),
                  pl.BlockSpec((_BCE_BR, _COLS), lambda i: (i, 0))],
        out_specs=[pl.BlockSpec((_BCE_BR, _COLS), lambda i: (i, 0)),
                   pl.BlockSpec(memory_space=pltpu.SMEM)],
        out_shape=[jax.ShapeDtypeStruct((_ROWS, _COLS), jnp.float32),
                   jax.ShapeDtypeStruct((_BCE_GRID,), jnp.float32)],
    )(x2d, t2d)


def _zero_hist(hist_v, words):
    zeros16 = jnp.zeros((_L,), jnp.int32)

    def zbody(i, c):
        for j in range(8):
            hist_v[pl.ds(i * 8 * _L + j * _L, _L)] = zeros16
        return c

    lax.fori_loop(0, words // (8 * _L), zbody, 0)


def _mesh():
    return plsc.VectorSubcoreMesh(
        core_axis_name="c", subcore_axis_name="s", num_cores=_NC)


def _wid():
    return lax.axis_index("c") * _NS + lax.axis_index("s")


def _make_pass(level):
    nbins = {1: _B1, 2: _B2, 3: _B3}[level]
    words = nbins * _L

    @functools.partial(
        pl.kernel,
        mesh=_mesh(),
        out_type=[jax.ShapeDtypeStruct((_NW * words,), jnp.int32),
                  jax.ShapeDtypeStruct((_NW * _L,), jnp.float32)],
        scratch_types=[pltpu.VMEM((words,), jnp.int32),
                       pltpu.VMEM((_CHUNK,), jnp.float32),
                       pltpu.VMEM((_CHUNK,), jnp.float32),
                       pltpu.VMEM((_L,), jnp.float32),
                       pltpu.VMEM((2 * _L,), jnp.int32),
                       pltpu.SemaphoreType.DMA,
                       pltpu.SemaphoreType.DMA],
        compiler_params=pltpu.CompilerParams(needs_layout_passes=False),
    )
    def apass(raw_hbm, sel_hbm, hist_out, acc_out, hist_v, buf_a, buf_b,
              acc_v, sel_v, sem_a, sem_b):
        wid = _wid()
        base = wid * _PER_W
        _zero_hist(hist_v, words)
        pltpu.sync_copy(sel_hbm, sel_v)
        sel_a = sel_v[pl.ds(0, _L)]
        sel_b = sel_v[pl.ds(_L, _L)]
        lane = lax.iota(jnp.int32, _L)
        ones = jnp.ones((_L,), jnp.int32)

        def chunk_src(c):
            return raw_hbm.at[pl.ds(base + c * _CHUNK, _CHUNK)]

        def process(buf, acc):
            def vbody(i, acc2):
                for j in range(_UNROLL):
                    v = buf[pl.ds((i * _UNROLL + j) * _L, _L)]
                    key = plsc.bitcast(v, jnp.int32)
                    if level == 1:
                        b = jax.lax.shift_right_logical(key, 20)
                        idx = b * _L + lane
                        plsc.addupdate_scatter(hist_v, [idx], ones)
                    elif level == 2:
                        f = jax.lax.shift_right_logical(key, 20)
                        m_eq = f == sel_a
                        m_gt = f > sel_a
                        b = jnp.bitwise_and(
                            jax.lax.shift_right_logical(key, 8), 0xFFF)
                        idx = b * _L + lane
                        plsc.addupdate_scatter(hist_v, [idx], ones,
                                               mask=m_eq)
                        acc2 = acc2 + jnp.where(m_gt, v, 0.0)
                    else:
                        p2 = jax.lax.shift_right_logical(key, 8)
                        m_eq = p2 == sel_a
                        m_gt = jnp.logical_and(p2 > sel_a, p2 < sel_b)
                        b = jnp.bitwise_and(key, 0xFF)
                        idx = b * _L + lane
                        plsc.addupdate_scatter(hist_v, [idx], ones,
                                               mask=m_eq)
                        acc2 = acc2 + jnp.where(m_gt, v, 0.0)
                return acc2

            return lax.fori_loop(0, _CHUNK // (_L * _UNROLL), vbody, acc)

        pltpu.async_copy(chunk_src(0), buf_a, sem_a)

        def outer(o, acc):
            c0 = 2 * o
            pltpu.async_copy(chunk_src(c0 + 1), buf_b, sem_b)
            pltpu.make_async_copy(chunk_src(c0), buf_a, sem_a).wait()
            acc = process(buf_a, acc)
            nxt = jnp.where(c0 + 2 < _NCHUNK, c0 + 2, 0)
            pltpu.async_copy(chunk_src(nxt), buf_a, sem_a)
            pltpu.make_async_copy(chunk_src(c0 + 1), buf_b, sem_b).wait()
            return process(buf_b, acc)

        acc = lax.fori_loop(0, _NCHUNK // 2, outer,
                            jnp.zeros((_L,), jnp.float32))
        pltpu.make_async_copy(chunk_src(0), buf_a, sem_a).wait()
        pltpu.sync_copy(hist_v, hist_out.at[pl.ds(wid * words, words)])
        acc_v[...] = acc
        pltpu.sync_copy(acc_v, acc_out.at[pl.ds(wid * _L, _L)])

    return apass


_pass1 = _make_pass(1)
_pass2 = _make_pass(2)
_pass3 = _make_pass(3)


def _select(counts, kk):
    c = counts
    above = jnp.cumsum(c[::-1])[::-1] - c
    cond = jnp.logical_and(above < kk, above + c >= kk)
    b = jnp.argmax(cond).astype(jnp.int32)
    return b, above[b]


def kernel(input, target, cur_epoch):
    x = input.reshape(_ROWS, _COLS)
    t = target.reshape(_ROWS, _COLS)
    raw2d, psums = _bce(x, t)
    raw = raw2d.reshape(-1)
    warm = jnp.sum(psums) / _N

    sel1 = jnp.zeros((2 * _L,), jnp.int32)
    h1, _ = _pass1(raw, sel1)
    c1 = jnp.sum(h1.reshape(_NW, _B1, _L), axis=(0, 2))
    b1, s1 = _select(c1, _K)
    k1 = _K - s1

    sel2 = jnp.concatenate([
        jnp.broadcast_to(b1, (_L,)),
        jnp.zeros((_L,), jnp.int32),
    ]).astype(jnp.int32)
    h2, a2 = _pass2(raw, sel2)
    c2 = jnp.sum(h2.reshape(_NW, _B2, _L), axis=(0, 2))
    sum_hi2 = jnp.sum(a2)
    b2, s2 = _select(c2, k1)
    k2 = k1 - s2

    pref2 = b1 * 4096 + b2
    hi2 = (b1 + 1) * 4096
    sel3 = jnp.concatenate([
        jnp.broadcast_to(pref2, (_L,)),
        jnp.broadcast_to(hi2, (_L,)),
    ]).astype(jnp.int32)
    h3, a3 = _pass3(raw, sel3)
    c3 = jnp.sum(h3.reshape(_NW, _B3, _L), axis=(0, 2))
    sum_hi3 = jnp.sum(a3)
    b3, s3 = _select(c3, k2)

    cnt_gt = s1 + s2 + s3
    base_key = pref2 * 256
    keys3 = base_key + jnp.arange(_B3, dtype=jnp.int32)
    vals3 = lax.bitcast_convert_type(keys3, jnp.float32)
    bins3 = jnp.arange(_B3, dtype=jnp.int32)
    sum_rec = jnp.sum(
        jnp.where(bins3 > b3, c3.astype(jnp.float32) * vals3, 0.0))
    tau = lax.bitcast_convert_type((base_key + b3).astype(jnp.int32),
                                   jnp.float32)
    sum_gt = sum_hi2 + sum_hi3 + sum_rec
    ties = (_K - cnt_gt).astype(jnp.float32)
    mean_top = (sum_gt + ties * tau) / _K
    return jnp.where(cur_epoch < _START_WARM, warm, mean_top)


# speedup vs baseline: 14.3024x; 1.2432x over previous
import functools

import jax
import jax.numpy as jnp
from jax import lax
from jax.experimental import pallas as pl
from jax.experimental.pallas import tpu as pltpu
from jax.experimental.pallas import tpu_sc as plsc

_START_WARM = 5000
_END_WARM = 15000
_TOP_P = 0.15
_CUR_EPOCH_VALUE = 10000

_ROWS, _COLS = 4096, 1024
_N = _ROWS * _COLS

if _CUR_EPOCH_VALUE > _END_WARM:
    _THIS_P = _TOP_P
else:
    _THIS_P = _TOP_P + (1.0 - _TOP_P) * (
        (_END_WARM - _CUR_EPOCH_VALUE) / (_END_WARM - _START_WARM))
_K = int(_N * _THIS_P)

_NC, _NS, _L = 2, 16, 16
_NW = _NC * _NS
_PER_W = _N // _NW
_CHUNK = 16384
_NCHUNK = _PER_W // _CHUNK
_UNROLL = 8

_B1, _B2, _B3 = 2048, 4096, 256


def _bce_body(x_ref, t_ref, raw_ref, psum_ref):
    x = x_ref[...]
    t = t_ref[...]
    raw = jnp.maximum(x, 0.0) - x * t + jnp.log1p(jnp.exp(-jnp.abs(x)))
    raw_ref[...] = raw
    psum_ref[pl.program_id(0)] = jnp.sum(raw)


_BCE_GRID = 8
_BCE_BR = _ROWS // _BCE_GRID


def _bce(x2d, t2d):
    return pl.pallas_call(
        _bce_body,
        grid=(_BCE_GRID,),
        in_specs=[pl.BlockSpec((_BCE_BR, _COLS), lambda i: (i, 0)),
                  pl.BlockSpec((_BCE_BR, _COLS), lambda i: (i, 0))],
        out_specs=[pl.BlockSpec((_BCE_BR, _COLS), lambda i: (i, 0)),
                   pl.BlockSpec(memory_space=pltpu.SMEM)],
        out_shape=[jax.ShapeDtypeStruct((_ROWS, _COLS), jnp.float32),
                   jax.ShapeDtypeStruct((_BCE_GRID,), jnp.float32)],
    )(x2d, t2d)


def _zero_hist(hist_v, words):
    zeros16 = jnp.zeros((_L,), jnp.int32)

    def zbody(i, c):
        for j in range(8):
            hist_v[pl.ds(i * 8 * _L + j * _L, _L)] = zeros16
        return c

    lax.fori_loop(0, words // (8 * _L), zbody, 0)


def _mesh():
    return plsc.VectorSubcoreMesh(
        core_axis_name="c", subcore_axis_name="s", num_cores=_NC)


def _wid():
    return lax.axis_index("c") * _NS + lax.axis_index("s")


def _make_pass(level):
    nbins = {1: _B1, 2: _B2, 3: _B3}[level]
    words = nbins * _L

    @functools.partial(
        pl.kernel,
        mesh=_mesh(),
        out_type=[jax.ShapeDtypeStruct((_NW * nbins,), jnp.int32),
                  jax.ShapeDtypeStruct((_NW * _L,), jnp.float32)],
        scratch_types=[pltpu.VMEM((words,), jnp.int32),
                       pltpu.VMEM((_CHUNK,), jnp.float32),
                       pltpu.VMEM((_CHUNK,), jnp.float32),
                       pltpu.VMEM((_L,), jnp.float32),
                       pltpu.VMEM((2 * _L,), jnp.int32),
                       pltpu.SemaphoreType.DMA,
                       pltpu.SemaphoreType.DMA],
        compiler_params=pltpu.CompilerParams(needs_layout_passes=False),
    )
    def apass(raw_hbm, sel_hbm, hist_out, acc_out, hist_v, buf_a, buf_b,
              acc_v, sel_v, sem_a, sem_b):
        wid = _wid()
        base = wid * _PER_W
        _zero_hist(hist_v, words)
        pltpu.sync_copy(sel_hbm, sel_v)
        sel_a = sel_v[pl.ds(0, _L)]
        sel_b = sel_v[pl.ds(_L, _L)]
        lane_base = lax.iota(jnp.int32, _L) * nbins
        ones = jnp.ones((_L,), jnp.int32)

        def chunk_src(c):
            return raw_hbm.at[pl.ds(base + c * _CHUNK, _CHUNK)]

        def process(buf, acc):
            def vbody(i, acc2):
                for j in range(_UNROLL):
                    v = buf[pl.ds((i * _UNROLL + j) * _L, _L)]
                    key = plsc.bitcast(v, jnp.int32)
                    if level == 1:
                        b = jax.lax.shift_right_logical(key, 20)
                        plsc.addupdate_scatter(hist_v, [lane_base + b],
                                               ones)
                    elif level == 2:
                        f = jax.lax.shift_right_logical(key, 20)
                        m_eq = f == sel_a
                        m_gt = f > sel_a
                        b = jnp.bitwise_and(
                            jax.lax.shift_right_logical(key, 8), 0xFFF)
                        plsc.addupdate_scatter(hist_v, [lane_base + b],
                                               ones, mask=m_eq)
                        acc2 = acc2 + jnp.where(m_gt, v, 0.0)
                    else:
                        p2 = jax.lax.shift_right_logical(key, 8)
                        m_eq = p2 == sel_a
                        m_gt = jnp.logical_and(p2 > sel_a, p2 < sel_b)
                        b = jnp.bitwise_and(key, 0xFF)
                        plsc.addupdate_scatter(hist_v, [lane_base + b],
                                               ones, mask=m_eq)
                        acc2 = acc2 + jnp.where(m_gt, v, 0.0)
                return acc2

            return lax.fori_loop(0, _CHUNK // (_L * _UNROLL), vbody, acc)

        pltpu.async_copy(chunk_src(0), buf_a, sem_a)

        def outer(o, acc):
            c0 = 2 * o
            pltpu.async_copy(chunk_src(c0 + 1), buf_b, sem_b)
            pltpu.make_async_copy(chunk_src(c0), buf_a, sem_a).wait()
            acc = process(buf_a, acc)
            nxt = jnp.where(c0 + 2 < _NCHUNK, c0 + 2, 0)
            pltpu.async_copy(chunk_src(nxt), buf_a, sem_a)
            pltpu.make_async_copy(chunk_src(c0 + 1), buf_b, sem_b).wait()
            return process(buf_b, acc)

        acc = lax.fori_loop(0, _NCHUNK // 2, outer,
                            jnp.zeros((_L,), jnp.float32))
        pltpu.make_async_copy(chunk_src(0), buf_a, sem_a).wait()

        def fold(j, c):
            s = hist_v[pl.ds(j * _L, _L)]
            for l in range(1, _L):
                s = s + hist_v[pl.ds(l * nbins + j * _L, _L)]
            hist_v[pl.ds(j * _L, _L)] = s
            return c

        lax.fori_loop(0, nbins // _L, fold, 0)
        pltpu.sync_copy(hist_v.at[pl.ds(0, nbins)],
                        hist_out.at[pl.ds(wid * nbins, nbins)])
        acc_v[...] = acc
        pltpu.sync_copy(acc_v, acc_out.at[pl.ds(wid * _L, _L)])

    return apass


_pass1 = _make_pass(1)
_pass2 = _make_pass(2)
_pass3 = _make_pass(3)


def _select(counts, kk):
    c = counts
    above = jnp.cumsum(c[::-1])[::-1] - c
    cond = jnp.logical_and(above < kk, above + c >= kk)
    b = jnp.argmax(cond).astype(jnp.int32)
    return b, above[b]


def kernel(input, target, cur_epoch):
    x = input.reshape(_ROWS, _COLS)
    t = target.reshape(_ROWS, _COLS)
    raw2d, psums = _bce(x, t)
    raw = raw2d.reshape(-1)
    warm = jnp.sum(psums) / _N

    sel1 = jnp.zeros((2 * _L,), jnp.int32)
    h1, _ = _pass1(raw, sel1)
    c1 = jnp.sum(h1.reshape(_NW, _B1), axis=0)
    b1, s1 = _select(c1, _K)
    k1 = _K - s1

    sel2 = jnp.concatenate([
        jnp.broadcast_to(b1, (_L,)),
        jnp.zeros((_L,), jnp.int32),
    ]).astype(jnp.int32)
    h2, a2 = _pass2(raw, sel2)
    c2 = jnp.sum(h2.reshape(_NW, _B2), axis=0)
    sum_hi2 = jnp.sum(a2)
    b2, s2 = _select(c2, k1)
    k2 = k1 - s2

    pref2 = b1 * 4096 + b2
    hi2 = (b1 + 1) * 4096
    sel3 = jnp.concatenate([
        jnp.broadcast_to(pref2, (_L,)),
        jnp.broadcast_to(hi2, (_L,)),
    ]).astype(jnp.int32)
    h3, a3 = _pass3(raw, sel3)
    c3 = jnp.sum(h3.reshape(_NW, _B3), axis=0)
    sum_hi3 = jnp.sum(a3)
    b3, s3 = _select(c3, k2)

    cnt_gt = s1 + s2 + s3
    base_key = pref2 * 256
    keys3 = base_key + jnp.arange(_B3, dtype=jnp.int32)
    vals3 = lax.bitcast_convert_type(keys3, jnp.float32)
    bins3 = jnp.arange(_B3, dtype=jnp.int32)
    sum_rec = jnp.sum(
        jnp.where(bins3 > b3, c3.astype(jnp.float32) * vals3, 0.0))
    tau = lax.bitcast_convert_type((base_key + b3).astype(jnp.int32),
                                   jnp.float32)
    sum_gt = sum_hi2 + sum_hi3 + sum_rec
    ties = (_K - cnt_gt).astype(jnp.float32)
    mean_top = (sum_gt + ties * tau) / _K
    return jnp.where(cur_epoch < _START_WARM, warm, mean_top)


# speedup vs baseline: 15.2826x; 1.0685x over previous
import functools

import jax
import jax.numpy as jnp
from jax import lax
from jax.experimental import pallas as pl
from jax.experimental.pallas import tpu as pltpu
from jax.experimental.pallas import tpu_sc as plsc

_START_WARM = 5000
_END_WARM = 15000
_TOP_P = 0.15
_CUR_EPOCH_VALUE = 10000

_ROWS, _COLS = 4096, 1024
_N = _ROWS * _COLS

if _CUR_EPOCH_VALUE > _END_WARM:
    _THIS_P = _TOP_P
else:
    _THIS_P = _TOP_P + (1.0 - _TOP_P) * (
        (_END_WARM - _CUR_EPOCH_VALUE) / (_END_WARM - _START_WARM))
_K = int(_N * _THIS_P)

_NC, _NS, _L = 2, 16, 16
_NW = _NC * _NS
_PER_W = _N // _NW
_CHUNK = 16384
_NCHUNK = _PER_W // _CHUNK
_UNROLL = 8

_B1, _B2, _B3 = 2048, 4096, 256


def _bce_body(x_ref, t_ref, raw_ref, psum_ref):
    x = x_ref[...]
    t = t_ref[...]
    raw = jnp.maximum(x, 0.0) - x * t + jnp.log1p(jnp.exp(-jnp.abs(x)))
    raw_ref[...] = raw
    psum_ref[pl.program_id(0)] = jnp.sum(raw)


_BCE_GRID = 8
_BCE_BR = _ROWS // _BCE_GRID


def _bce(x2d, t2d):
    return pl.pallas_call(
        _bce_body,
        grid=(_BCE_GRID,),
        in_specs=[pl.BlockSpec((_BCE_BR, _COLS), lambda i: (i, 0)),
                  pl.BlockSpec((_BCE_BR, _COLS), lambda i: (i, 0))],
        out_specs=[pl.BlockSpec((_BCE_BR, _COLS), lambda i: (i, 0)),
                   pl.BlockSpec(memory_space=pltpu.SMEM)],
        out_shape=[jax.ShapeDtypeStruct((_ROWS, _COLS), jnp.float32),
                   jax.ShapeDtypeStruct((_BCE_GRID,), jnp.float32)],
    )(x2d, t2d)


def _zero_hist(hist_v, words):
    zeros16 = jnp.zeros((_L,), jnp.int32)

    def zbody(i, c):
        for j in range(8):
            hist_v[pl.ds(i * 8 * _L + j * _L, _L)] = zeros16
        return c

    lax.fori_loop(0, words // (8 * _L), zbody, 0)


def _mesh():
    return plsc.VectorSubcoreMesh(
        core_axis_name="c", subcore_axis_name="s", num_cores=_NC)


def _wid():
    return lax.axis_index("c") * _NS + lax.axis_index("s")


def _make_pass(level):
    nbins = {1: _B1, 2: _B2, 3: _B3}[level]
    words = nbins

    @functools.partial(
        pl.kernel,
        mesh=_mesh(),
        out_type=[jax.ShapeDtypeStruct((_NW * nbins,), jnp.int32),
                  jax.ShapeDtypeStruct((_NW * _L,), jnp.float32)],
        scratch_types=[pltpu.VMEM((words,), jnp.int32),
                       pltpu.VMEM((_CHUNK,), jnp.float32),
                       pltpu.VMEM((_CHUNK,), jnp.float32),
                       pltpu.VMEM((_L,), jnp.float32),
                       pltpu.VMEM((2 * _L,), jnp.int32),
                       pltpu.SemaphoreType.DMA,
                       pltpu.SemaphoreType.DMA],
        compiler_params=pltpu.CompilerParams(needs_layout_passes=False),
    )
    def apass(raw_hbm, sel_hbm, hist_out, acc_out, hist_v, buf_a, buf_b,
              acc_v, sel_v, sem_a, sem_b):
        wid = _wid()
        base = wid * _PER_W
        _zero_hist(hist_v, words)
        pltpu.sync_copy(sel_hbm, sel_v)
        sel_a = sel_v[pl.ds(0, _L)]
        sel_b = sel_v[pl.ds(_L, _L)]
        ones = jnp.ones((_L,), jnp.int32)

        def chunk_src(c):
            return raw_hbm.at[pl.ds(base + c * _CHUNK, _CHUNK)]

        def process(buf, acc):
            def vbody(i, acc2):
                for j in range(_UNROLL):
                    v = buf[pl.ds((i * _UNROLL + j) * _L, _L)]
                    key = plsc.bitcast(v, jnp.int32)
                    if level == 1:
                        b = jax.lax.shift_right_logical(key, 20)
                        plsc.addupdate_scatter(hist_v, [b], ones)
                    elif level == 2:
                        f = jax.lax.shift_right_logical(key, 20)
                        m_eq = f == sel_a
                        m_gt = f > sel_a
                        b = jnp.bitwise_and(
                            jax.lax.shift_right_logical(key, 8), 0xFFF)
                        plsc.addupdate_scatter(hist_v, [b], ones,
                                               mask=m_eq)
                        acc2 = acc2 + jnp.where(m_gt, v, 0.0)
                    else:
                        p2 = jax.lax.shift_right_logical(key, 8)
                        m_eq = p2 == sel_a
                        m_gt = jnp.logical_and(p2 > sel_a, p2 < sel_b)
                        b = jnp.bitwise_and(key, 0xFF)
                        plsc.addupdate_scatter(hist_v, [b], ones,
                                               mask=m_eq)
                        acc2 = acc2 + jnp.where(m_gt, v, 0.0)
                return acc2

            return lax.fori_loop(0, _CHUNK // (_L * _UNROLL), vbody, acc)

        pltpu.async_copy(chunk_src(0), buf_a, sem_a)

        def outer(o, acc):
            c0 = 2 * o
            pltpu.async_copy(chunk_src(c0 + 1), buf_b, sem_b)
            pltpu.make_async_copy(chunk_src(c0), buf_a, sem_a).wait()
            acc = process(buf_a, acc)
            nxt = jnp.where(c0 + 2 < _NCHUNK, c0 + 2, 0)
            pltpu.async_copy(chunk_src(nxt), buf_a, sem_a)
            pltpu.make_async_copy(chunk_src(c0 + 1), buf_b, sem_b).wait()
            return process(buf_b, acc)

        acc = lax.fori_loop(0, _NCHUNK // 2, outer,
                            jnp.zeros((_L,), jnp.float32))
        pltpu.make_async_copy(chunk_src(0), buf_a, sem_a).wait()
        pltpu.sync_copy(hist_v.at[pl.ds(0, nbins)],
                        hist_out.at[pl.ds(wid * nbins, nbins)])
        acc_v[...] = acc
        pltpu.sync_copy(acc_v, acc_out.at[pl.ds(wid * _L, _L)])

    return apass


_pass1 = _make_pass(1)
_pass2 = _make_pass(2)
_pass3 = _make_pass(3)


def _select(counts, kk):
    c = counts
    above = jnp.cumsum(c[::-1])[::-1] - c
    cond = jnp.logical_and(above < kk, above + c >= kk)
    b = jnp.argmax(cond).astype(jnp.int32)
    return b, above[b]


def kernel(input, target, cur_epoch):
    x = input.reshape(_ROWS, _COLS)
    t = target.reshape(_ROWS, _COLS)
    raw2d, psums = _bce(x, t)
    raw = raw2d.reshape(-1)
    warm = jnp.sum(psums) / _N

    sel1 = jnp.zeros((2 * _L,), jnp.int32)
    h1, _ = _pass1(raw, sel1)
    c1 = jnp.sum(h1.reshape(_NW, _B1), axis=0)
    b1, s1 = _select(c1, _K)
    k1 = _K - s1

    sel2 = jnp.concatenate([
        jnp.broadcast_to(b1, (_L,)),
        jnp.zeros((_L,), jnp.int32),
    ]).astype(jnp.int32)
    h2, a2 = _pass2(raw, sel2)
    c2 = jnp.sum(h2.reshape(_NW, _B2), axis=0)
    sum_hi2 = jnp.sum(a2)
    b2, s2 = _select(c2, k1)
    k2 = k1 - s2

    pref2 = b1 * 4096 + b2
    hi2 = (b1 + 1) * 4096
    sel3 = jnp.concatenate([
        jnp.broadcast_to(pref2, (_L,)),
        jnp.broadcast_to(hi2, (_L,)),
    ]).astype(jnp.int32)
    h3, a3 = _pass3(raw, sel3)
    c3 = jnp.sum(h3.reshape(_NW, _B3), axis=0)
    sum_hi3 = jnp.sum(a3)
    b3, s3 = _select(c3, k2)

    cnt_gt = s1 + s2 + s3
    base_key = pref2 * 256
    keys3 = base_key + jnp.arange(_B3, dtype=jnp.int32)
    vals3 = lax.bitcast_convert_type(keys3, jnp.float32)
    bins3 = jnp.arange(_B3, dtype=jnp.int32)
    sum_rec = jnp.sum(
        jnp.where(bins3 > b3, c3.astype(jnp.float32) * vals3, 0.0))
    tau = lax.bitcast_convert_type((base_key + b3).astype(jnp.int32),
                                   jnp.float32)
    sum_gt = sum_hi2 + sum_hi3 + sum_rec
    ties = (_K - cnt_gt).astype(jnp.float32)
    mean_top = (sum_gt + ties * tau) / _K
    return jnp.where(cur_epoch < _START_WARM, warm, mean_top)


# speedup vs baseline: 18.2467x; 1.1940x over previous
import functools

import jax
import jax.numpy as jnp
from jax import lax
from jax.experimental import pallas as pl
from jax.experimental.pallas import tpu as pltpu
from jax.experimental.pallas import tpu_sc as plsc

_START_WARM = 5000
_END_WARM = 15000
_TOP_P = 0.15
_CUR_EPOCH_VALUE = 10000

_ROWS, _COLS = 4096, 1024
_N = _ROWS * _COLS

if _CUR_EPOCH_VALUE > _END_WARM:
    _THIS_P = _TOP_P
else:
    _THIS_P = _TOP_P + (1.0 - _TOP_P) * (
        (_END_WARM - _CUR_EPOCH_VALUE) / (_END_WARM - _START_WARM))
_K = int(_N * _THIS_P)

_NC, _NS, _L = 2, 16, 16
_NW = _NC * _NS
_PER_W = _N // _NW
_CHUNK = 16384
_NCHUNK = _PER_W // _CHUNK
_UNROLL = 8

_B1, _B2 = 32768, 65536


def _bce_body(x_ref, t_ref, raw_ref, psum_ref):
    x = x_ref[...]
    t = t_ref[...]
    raw = jnp.maximum(x, 0.0) - x * t + jnp.log1p(jnp.exp(-jnp.abs(x)))
    raw_ref[...] = raw
    psum_ref[pl.program_id(0)] = jnp.sum(raw)


_BCE_GRID = 8
_BCE_BR = _ROWS // _BCE_GRID


def _bce(x2d, t2d):
    return pl.pallas_call(
        _bce_body,
        grid=(_BCE_GRID,),
        in_specs=[pl.BlockSpec((_BCE_BR, _COLS), lambda i: (i, 0)),
                  pl.BlockSpec((_BCE_BR, _COLS), lambda i: (i, 0))],
        out_specs=[pl.BlockSpec((_BCE_BR, _COLS), lambda i: (i, 0)),
                   pl.BlockSpec(memory_space=pltpu.SMEM)],
        out_shape=[jax.ShapeDtypeStruct((_ROWS, _COLS), jnp.float32),
                   jax.ShapeDtypeStruct((_BCE_GRID,), jnp.float32)],
    )(x2d, t2d)


def _zero_hist(hist_v, words):
    zeros16 = jnp.zeros((_L,), jnp.int32)

    def zbody(i, c):
        for j in range(8):
            hist_v[pl.ds(i * 8 * _L + j * _L, _L)] = zeros16
        return c

    lax.fori_loop(0, words // (8 * _L), zbody, 0)


def _mesh():
    return plsc.VectorSubcoreMesh(
        core_axis_name="c", subcore_axis_name="s", num_cores=_NC)


def _wid():
    return lax.axis_index("c") * _NS + lax.axis_index("s")


def _make_pass(level):
    nbins = _B1 if level == 1 else _B2
    words = nbins

    @functools.partial(
        pl.kernel,
        mesh=_mesh(),
        out_type=[jax.ShapeDtypeStruct((_NW * nbins,), jnp.int32),
                  jax.ShapeDtypeStruct((_NW * _L,), jnp.float32)],
        scratch_types=[pltpu.VMEM((words,), jnp.int32),
                       pltpu.VMEM((_CHUNK,), jnp.float32),
                       pltpu.VMEM((_CHUNK,), jnp.float32),
                       pltpu.VMEM((_L,), jnp.float32),
                       pltpu.VMEM((2 * _L,), jnp.int32),
                       pltpu.SemaphoreType.DMA,
                       pltpu.SemaphoreType.DMA],
        compiler_params=pltpu.CompilerParams(needs_layout_passes=False),
    )
    def apass(raw_hbm, sel_hbm, hist_out, acc_out, hist_v, buf_a, buf_b,
              acc_v, sel_v, sem_a, sem_b):
        wid = _wid()
        base = wid * _PER_W
        _zero_hist(hist_v, words)
        pltpu.sync_copy(sel_hbm, sel_v)
        sel_a = sel_v[pl.ds(0, _L)]
        ones = jnp.ones((_L,), jnp.int32)

        def chunk_src(c):
            return raw_hbm.at[pl.ds(base + c * _CHUNK, _CHUNK)]

        def process(buf, acc):
            def vbody(i, acc2):
                for j in range(_UNROLL):
                    v = buf[pl.ds((i * _UNROLL + j) * _L, _L)]
                    key = plsc.bitcast(v, jnp.int32)
                    if level == 1:
                        b = jax.lax.shift_right_logical(key, 16)
                        plsc.addupdate_scatter(hist_v, [b], ones)
                    else:
                        f = jax.lax.shift_right_logical(key, 16)
                        m_eq = f == sel_a
                        m_gt = f > sel_a
                        b = jnp.bitwise_and(key, 0xFFFF)
                        plsc.addupdate_scatter(hist_v, [b], ones,
                                               mask=m_eq)
                        acc2 = acc2 + jnp.where(m_gt, v, 0.0)
                return acc2

            return lax.fori_loop(0, _CHUNK // (_L * _UNROLL), vbody, acc)

        pltpu.async_copy(chunk_src(0), buf_a, sem_a)

        def outer(o, acc):
            c0 = 2 * o
            pltpu.async_copy(chunk_src(c0 + 1), buf_b, sem_b)
            pltpu.make_async_copy(chunk_src(c0), buf_a, sem_a).wait()
            acc = process(buf_a, acc)
            nxt = jnp.where(c0 + 2 < _NCHUNK, c0 + 2, 0)
            pltpu.async_copy(chunk_src(nxt), buf_a, sem_a)
            pltpu.make_async_copy(chunk_src(c0 + 1), buf_b, sem_b).wait()
            return process(buf_b, acc)

        acc = lax.fori_loop(0, _NCHUNK // 2, outer,
                            jnp.zeros((_L,), jnp.float32))
        pltpu.make_async_copy(chunk_src(0), buf_a, sem_a).wait()
        pltpu.sync_copy(hist_v.at[pl.ds(0, nbins)],
                        hist_out.at[pl.ds(wid * nbins, nbins)])
        acc_v[...] = acc
        pltpu.sync_copy(acc_v, acc_out.at[pl.ds(wid * _L, _L)])

    return apass


_pass1 = _make_pass(1)
_pass2 = _make_pass(2)


def _select(counts, kk):
    c = counts
    above = jnp.cumsum(c[::-1])[::-1] - c
    cond = jnp.logical_and(above < kk, above + c >= kk)
    b = jnp.argmax(cond).astype(jnp.int32)
    return b, above[b]


def kernel(input, target, cur_epoch):
    x = input.reshape(_ROWS, _COLS)
    t = target.reshape(_ROWS, _COLS)
    raw2d, psums = _bce(x, t)
    raw = raw2d.reshape(-1)
    warm = jnp.sum(psums) / _N

    sel1 = jnp.zeros((2 * _L,), jnp.int32)
    h1, _ = _pass1(raw, sel1)
    c1 = jnp.sum(h1.reshape(_NW, _B1), axis=0)
    b1, s1 = _select(c1, _K)
    k1 = _K - s1

    sel2 = jnp.concatenate([
        jnp.broadcast_to(b1, (_L,)),
        jnp.zeros((_L,), jnp.int32),
    ]).astype(jnp.int32)
    h2, a2 = _pass2(raw, sel2)
    c2 = jnp.sum(h2.reshape(_NW, _B2), axis=0)
    sum_hi = jnp.sum(a2)
    b2, s2 = _select(c2, k1)

    cnt_gt = s1 + s2
    base_key = b1 * _B2
    keys2 = base_key + jnp.arange(_B2, dtype=jnp.int32)
    vals2 = lax.bitcast_convert_type(keys2, jnp.float32)
    bins2 = jnp.arange(_B2, dtype=jnp.int32)
    sum_rec = jnp.sum(
        jnp.where(bins2 > b2, c2.astype(jnp.float32) * vals2, 0.0))
    tau = lax.bitcast_convert_type((base_key + b2).astype(jnp.int32),
                                   jnp.float32)
    sum_gt = sum_hi + sum_rec
    ties = (_K - cnt_gt).astype(jnp.float32)
    mean_top = (sum_gt + ties * tau) / _K
    return jnp.where(cur_epoch < _START_WARM, warm, mean_top)


# speedup vs baseline: 18.3164x; 1.0038x over previous
import functools

import jax
import jax.numpy as jnp
from jax import lax
from jax.experimental import pallas as pl
from jax.experimental.pallas import tpu as pltpu
from jax.experimental.pallas import tpu_sc as plsc

_START_WARM = 5000
_END_WARM = 15000
_TOP_P = 0.15
_CUR_EPOCH_VALUE = 10000

_ROWS, _COLS = 4096, 1024
_N = _ROWS * _COLS

if _CUR_EPOCH_VALUE > _END_WARM:
    _THIS_P = _TOP_P
else:
    _THIS_P = _TOP_P + (1.0 - _TOP_P) * (
        (_END_WARM - _CUR_EPOCH_VALUE) / (_END_WARM - _START_WARM))
_K = int(_N * _THIS_P)

_NC, _NS, _L = 2, 16, 16
_NW = _NC * _NS
_PER_W = _N // _NW
_CHUNK = 16384
_NCHUNK = _PER_W // _CHUNK
_UNROLL = 16

_B1, _B2 = 32768, 65536


def _bce_body(x_ref, t_ref, raw_ref, psum_ref):
    x = x_ref[...]
    t = t_ref[...]
    raw = jnp.maximum(x, 0.0) - x * t + jnp.log1p(jnp.exp(-jnp.abs(x)))
    raw_ref[...] = raw
    psum_ref[pl.program_id(0)] = jnp.sum(raw)


_BCE_GRID = 8
_BCE_BR = _ROWS // _BCE_GRID


def _bce(x2d, t2d):
    return pl.pallas_call(
        _bce_body,
        grid=(_BCE_GRID,),
        in_specs=[pl.BlockSpec((_BCE_BR, _COLS), lambda i: (i, 0)),
                  pl.BlockSpec((_BCE_BR, _COLS), lambda i: (i, 0))],
        out_specs=[pl.BlockSpec((_BCE_BR, _COLS), lambda i: (i, 0)),
                   pl.BlockSpec(memory_space=pltpu.SMEM)],
        out_shape=[jax.ShapeDtypeStruct((_ROWS, _COLS), jnp.float32),
                   jax.ShapeDtypeStruct((_BCE_GRID,), jnp.float32)],
    )(x2d, t2d)


def _zero_hist(hist_v, words):
    zeros16 = jnp.zeros((_L,), jnp.int32)

    def zbody(i, c):
        for j in range(8):
            hist_v[pl.ds(i * 8 * _L + j * _L, _L)] = zeros16
        return c

    lax.fori_loop(0, words // (8 * _L), zbody, 0)


def _mesh():
    return plsc.VectorSubcoreMesh(
        core_axis_name="c", subcore_axis_name="s", num_cores=_NC)


def _wid():
    return lax.axis_index("c") * _NS + lax.axis_index("s")


def _make_pass(level):
    nbins = _B1 if level == 1 else _B2
    words = nbins

    @functools.partial(
        pl.kernel,
        mesh=_mesh(),
        out_type=[jax.ShapeDtypeStruct((_NW * nbins,), jnp.int32),
                  jax.ShapeDtypeStruct((_NW * _L,), jnp.float32)],
        scratch_types=[pltpu.VMEM((words,), jnp.int32),
                       pltpu.VMEM((_CHUNK,), jnp.float32),
                       pltpu.VMEM((_CHUNK,), jnp.float32),
                       pltpu.VMEM((_L,), jnp.float32),
                       pltpu.VMEM((2 * _L,), jnp.int32),
                       pltpu.SemaphoreType.DMA,
                       pltpu.SemaphoreType.DMA],
        compiler_params=pltpu.CompilerParams(needs_layout_passes=False),
    )
    def apass(raw_hbm, sel_hbm, hist_out, acc_out, hist_v, buf_a, buf_b,
              acc_v, sel_v, sem_a, sem_b):
        wid = _wid()
        base = wid * _PER_W
        _zero_hist(hist_v, words)
        pltpu.sync_copy(sel_hbm, sel_v)
        sel_a = sel_v[pl.ds(0, _L)]
        ones = jnp.ones((_L,), jnp.int32)

        def chunk_src(c):
            return raw_hbm.at[pl.ds(base + c * _CHUNK, _CHUNK)]

        def process(buf, acc):
            def vbody(i, acc2):
                for j in range(_UNROLL):
                    v = buf[pl.ds((i * _UNROLL + j) * _L, _L)]
                    key = plsc.bitcast(v, jnp.int32)
                    if level == 1:
                        b = jax.lax.shift_right_logical(key, 16)
                        plsc.addupdate_scatter(hist_v, [b], ones)
                    else:
                        f = jax.lax.shift_right_logical(key, 16)
                        m_eq = f == sel_a
                        m_gt = f > sel_a
                        b = jnp.bitwise_and(key, 0xFFFF)
                        plsc.addupdate_scatter(hist_v, [b], ones,
                                               mask=m_eq)
                        acc2 = acc2 + jnp.where(m_gt, v, 0.0)
                return acc2

            return lax.fori_loop(0, _CHUNK // (_L * _UNROLL), vbody, acc)

        pltpu.async_copy(chunk_src(0), buf_a, sem_a)

        def outer(o, acc):
            c0 = 2 * o
            pltpu.async_copy(chunk_src(c0 + 1), buf_b, sem_b)
            pltpu.make_async_copy(chunk_src(c0), buf_a, sem_a).wait()
            acc = process(buf_a, acc)
            nxt = jnp.where(c0 + 2 < _NCHUNK, c0 + 2, 0)
            pltpu.async_copy(chunk_src(nxt), buf_a, sem_a)
            pltpu.make_async_copy(chunk_src(c0 + 1), buf_b, sem_b).wait()
            return process(buf_b, acc)

        acc = lax.fori_loop(0, _NCHUNK // 2, outer,
                            jnp.zeros((_L,), jnp.float32))
        pltpu.make_async_copy(chunk_src(0), buf_a, sem_a).wait()
        pltpu.sync_copy(hist_v.at[pl.ds(0, nbins)],
                        hist_out.at[pl.ds(wid * nbins, nbins)])
        acc_v[...] = acc
        pltpu.sync_copy(acc_v, acc_out.at[pl.ds(wid * _L, _L)])

    return apass


_pass1 = _make_pass(1)
_pass2 = _make_pass(2)


def _select(counts, kk):
    c = counts
    above = jnp.cumsum(c[::-1])[::-1] - c
    cond = jnp.logical_and(above < kk, above + c >= kk)
    b = jnp.argmax(cond).astype(jnp.int32)
    return b, above[b]


def kernel(input, target, cur_epoch):
    x = input.reshape(_ROWS, _COLS)
    t = target.reshape(_ROWS, _COLS)
    raw2d, psums = _bce(x, t)
    raw = raw2d.reshape(-1)
    warm = jnp.sum(psums) / _N

    sel1 = jnp.zeros((2 * _L,), jnp.int32)
    h1, _ = _pass1(raw, sel1)
    c1 = jnp.sum(h1.reshape(_NW, _B1), axis=0)
    b1, s1 = _select(c1, _K)
    k1 = _K - s1

    sel2 = jnp.concatenate([
        jnp.broadcast_to(b1, (_L,)),
        jnp.zeros((_L,), jnp.int32),
    ]).astype(jnp.int32)
    h2, a2 = _pass2(raw, sel2)
    c2 = jnp.sum(h2.reshape(_NW, _B2), axis=0)
    sum_hi = jnp.sum(a2)
    b2, s2 = _select(c2, k1)

    cnt_gt = s1 + s2
    base_key = b1 * _B2
    keys2 = base_key + jnp.arange(_B2, dtype=jnp.int32)
    vals2 = lax.bitcast_convert_type(keys2, jnp.float32)
    bins2 = jnp.arange(_B2, dtype=jnp.int32)
    sum_rec = jnp.sum(
        jnp.where(bins2 > b2, c2.astype(jnp.float32) * vals2, 0.0))
    tau = lax.bitcast_convert_type((base_key + b2).astype(jnp.int32),
                                   jnp.float32)
    sum_gt = sum_hi + sum_rec
    ties = (_K - cnt_gt).astype(jnp.float32)
    mean_top = (sum_gt + ties * tau) / _K
    return jnp.where(cur_epoch < _START_WARM, warm, mean_top)


# speedup vs baseline: 19.9959x; 1.0917x over previous
import functools

import jax
import jax.numpy as jnp
from jax import lax
from jax.experimental import pallas as pl
from jax.experimental.pallas import tpu as pltpu
from jax.experimental.pallas import tpu_sc as plsc

_START_WARM = 5000
_END_WARM = 15000
_TOP_P = 0.15
_CUR_EPOCH_VALUE = 10000

_ROWS, _COLS = 4096, 1024
_N = _ROWS * _COLS

if _CUR_EPOCH_VALUE > _END_WARM:
    _THIS_P = _TOP_P
else:
    _THIS_P = _TOP_P + (1.0 - _TOP_P) * (
        (_END_WARM - _CUR_EPOCH_VALUE) / (_END_WARM - _START_WARM))
_K = int(_N * _THIS_P)

_NC, _NS, _L = 2, 16, 16
_NW = _NC * _NS
_PER_W = _N // _NW
_CHUNK = 16384
_NCHUNK = _PER_W // _CHUNK
_UNROLL = 16

_B1, _B2 = 32768, 65536


def _bce_body(x_ref, t_ref, raw_ref, psum_ref):
    x = x_ref[...]
    t = t_ref[...]
    raw = jnp.maximum(x, 0.0) - x * t + jnp.log1p(jnp.exp(-jnp.abs(x)))
    raw_ref[...] = raw
    psum_ref[pl.program_id(0)] = jnp.sum(raw)


_BCE_GRID = 8
_BCE_BR = _ROWS // _BCE_GRID


def _bce(x2d, t2d):
    return pl.pallas_call(
        _bce_body,
        grid=(_BCE_GRID,),
        in_specs=[pl.BlockSpec((_BCE_BR, _COLS), lambda i: (i, 0)),
                  pl.BlockSpec((_BCE_BR, _COLS), lambda i: (i, 0))],
        out_specs=[pl.BlockSpec((_BCE_BR, _COLS), lambda i: (i, 0)),
                   pl.BlockSpec(memory_space=pltpu.SMEM)],
        out_shape=[jax.ShapeDtypeStruct((_ROWS, _COLS), jnp.float32),
                   jax.ShapeDtypeStruct((_BCE_GRID,), jnp.float32)],
    )(x2d, t2d)


def _zero_hist(hist_v, words):
    zeros16 = jnp.zeros((_L,), jnp.int32)

    def zbody(i, c):
        for j in range(8):
            hist_v[pl.ds(i * 8 * _L + j * _L, _L)] = zeros16
        return c

    lax.fori_loop(0, words // (8 * _L), zbody, 0)


def _mesh():
    return plsc.VectorSubcoreMesh(
        core_axis_name="c", subcore_axis_name="s", num_cores=_NC)


def _wid():
    return lax.axis_index("c") * _NS + lax.axis_index("s")


def _make_pass(level):
    nbins = _B1 if level == 1 else _B2
    words = nbins

    @functools.partial(
        pl.kernel,
        mesh=_mesh(),
        out_type=[jax.ShapeDtypeStruct((_NW * nbins,), jnp.int32),
                  jax.ShapeDtypeStruct((_NW * _L,), jnp.float32)],
        scratch_types=[pltpu.VMEM((words,), jnp.int32),
                       pltpu.VMEM((_CHUNK,), jnp.float32),
                       pltpu.VMEM((_CHUNK,), jnp.float32),
                       pltpu.VMEM((_L,), jnp.float32),
                       pltpu.VMEM((2 * _L,), jnp.int32),
                       pltpu.SemaphoreType.DMA,
                       pltpu.SemaphoreType.DMA],
        compiler_params=pltpu.CompilerParams(needs_layout_passes=False),
    )
    def apass(raw_hbm, sel_hbm, hist_out, acc_out, hist_v, buf_a, buf_b,
              acc_v, sel_v, sem_a, sem_b):
        wid = _wid()
        base = wid * _PER_W
        _zero_hist(hist_v, words)
        pltpu.sync_copy(sel_hbm, sel_v)
        sel_a = sel_v[pl.ds(0, _L)]
        ones = jnp.ones((_L,), jnp.int32)

        def chunk_src(c):
            return raw_hbm.at[pl.ds(base + c * _CHUNK, _CHUNK)]

        def process(buf, acc):
            def vbody(i, acc2):
                for j in range(_UNROLL):
                    v = buf[pl.ds((i * _UNROLL + j) * _L, _L)]
                    key = plsc.bitcast(v, jnp.int32)
                    if level == 1:
                        b = jax.lax.shift_right_logical(key, 16)
                        plsc.addupdate_scatter(hist_v, [b], ones)
                    else:
                        f = jax.lax.shift_right_logical(key, 16)
                        m_eq = f == sel_a
                        m_gt = f > sel_a
                        b = jnp.bitwise_and(key, 0xFFFF)
                        plsc.addupdate_scatter(hist_v, [b], ones,
                                               mask=m_eq)
                        acc2 = acc2 + jnp.where(m_gt, v, 0.0)
                return acc2

            return lax.fori_loop(0, _CHUNK // (_L * _UNROLL), vbody, acc)

        pltpu.async_copy(chunk_src(0), buf_a, sem_a)

        def outer(o, acc):
            c0 = 2 * o
            pltpu.async_copy(chunk_src(c0 + 1), buf_b, sem_b)
            pltpu.make_async_copy(chunk_src(c0), buf_a, sem_a).wait()
            acc = process(buf_a, acc)
            nxt = jnp.where(c0 + 2 < _NCHUNK, c0 + 2, 0)
            pltpu.async_copy(chunk_src(nxt), buf_a, sem_a)
            pltpu.make_async_copy(chunk_src(c0 + 1), buf_b, sem_b).wait()
            return process(buf_b, acc)

        acc = lax.fori_loop(0, _NCHUNK // 2, outer,
                            jnp.zeros((_L,), jnp.float32))
        pltpu.make_async_copy(chunk_src(0), buf_a, sem_a).wait()
        pltpu.sync_copy(hist_v.at[pl.ds(0, nbins)],
                        hist_out.at[pl.ds(wid * nbins, nbins)])
        acc_v[...] = acc
        pltpu.sync_copy(acc_v, acc_out.at[pl.ds(wid * _L, _L)])

    return apass


_pass1 = _make_pass(1)
_pass2 = _make_pass(2)


def _prefix_above(h, rows):
    c = jnp.sum(h.astype(jnp.float32), axis=0)
    iu = lax.broadcasted_iota(jnp.int32, (128, 128), 0)
    ju = lax.broadcasted_iota(jnp.int32, (128, 128), 1)
    upper_incl = (iu <= ju).astype(jnp.float32)
    row_incl = jnp.dot(c, upper_incl,
                       preferred_element_type=jnp.float32)
    s_row = row_incl[:, 127:128]
    ir = lax.broadcasted_iota(jnp.int32, (rows, rows), 0)
    jr = lax.broadcasted_iota(jnp.int32, (rows, rows), 1)
    lower_strict = (jr < ir).astype(jnp.float32)
    prev_rows = jnp.dot(lower_strict, s_row,
                        preferred_element_type=jnp.float32)
    incl = row_incl + prev_rows
    above = jnp.sum(c) - incl
    return c, above


def _flat_pos(rows, dtype):
    p = (lax.broadcasted_iota(jnp.int32, (rows, 128), 0) * 128
         + lax.broadcasted_iota(jnp.int32, (rows, 128), 1))
    return p.astype(dtype)


_S1R = _B1 // 128


def _sel1_body(h_ref, sel_ref, st_ref):
    c, above = _prefix_above(h_ref[...], _S1R)
    cond = jnp.logical_and(above < _K, above + c >= _K)
    pos = _flat_pos(_S1R, jnp.float32)
    b1 = jnp.sum(jnp.where(cond, pos, 0.0))
    s1 = jnp.sum(jnp.where(cond, above, 0.0))
    b1_i = b1.astype(jnp.int32)
    for j in range(2 * _L):
        sel_ref[j] = b1_i
    st_ref[0] = b1
    st_ref[1] = s1


def _sel1(h1):
    return pl.pallas_call(
        _sel1_body,
        out_shape=[jax.ShapeDtypeStruct((2 * _L,), jnp.int32),
                   jax.ShapeDtypeStruct((2,), jnp.float32)],
        out_specs=[pl.BlockSpec(memory_space=pltpu.SMEM),
                   pl.BlockSpec(memory_space=pltpu.SMEM)],
    )(h1.reshape(_NW, _S1R, 128))


_S2R = _B2 // 128


def _sel2_body(h_ref, a_ref, st_ref, ps_ref, ep_ref, out_ref):
    c, above = _prefix_above(h_ref[...], _S2R)
    b1 = st_ref[0]
    s1 = st_ref[1]
    k1 = _K - s1
    cond = jnp.logical_and(above < k1, above + c >= k1)
    pos_i = _flat_pos(_S2R, jnp.int32)
    pos_f = _flat_pos(_S2R, jnp.float32)
    b2 = jnp.sum(jnp.where(cond, pos_f, 0.0))
    s2 = jnp.sum(jnp.where(cond, above, 0.0))
    base = b1.astype(jnp.int32) * _B2
    vals = lax.bitcast_convert_type(base + pos_i, jnp.float32)
    sum_rec = jnp.sum(jnp.where(pos_f > b2, c * vals, 0.0))
    tau = jnp.sum(jnp.where(pos_f == b2, vals, 0.0))
    sum_hi = jnp.sum(a_ref[...])
    ties = _K - (s1 + s2)
    mean_top = (sum_hi + sum_rec + ties * tau) / _K
    warm = 0.0
    for j in range(_BCE_GRID):
        warm = warm + ps_ref[j]
    warm = warm / _N
    out_ref[0] = jnp.where(ep_ref[0] < _START_WARM, warm, mean_top)


def _sel2(h2, a2, st, psums, epoch):
    return pl.pallas_call(
        _sel2_body,
        in_specs=[pl.BlockSpec(memory_space=pltpu.VMEM),
                  pl.BlockSpec(memory_space=pltpu.VMEM),
                  pl.BlockSpec(memory_space=pltpu.SMEM),
                  pl.BlockSpec(memory_space=pltpu.SMEM),
                  pl.BlockSpec(memory_space=pltpu.SMEM)],
        out_shape=jax.ShapeDtypeStruct((1,), jnp.float32),
        out_specs=pl.BlockSpec(memory_space=pltpu.SMEM),
    )(h2.reshape(_NW, _S2R, 128), a2.reshape(4, 128), st, psums, epoch)


def kernel(input, target, cur_epoch):
    x = input.reshape(_ROWS, _COLS)
    t = target.reshape(_ROWS, _COLS)
    raw2d, psums = _bce(x, t)
    raw = raw2d.reshape(-1)

    sel0 = jnp.zeros((2 * _L,), jnp.int32)
    h1, _ = _pass1(raw, sel0)
    sel2, st = _sel1(h1)
    h2, a2 = _pass2(raw, sel2)
    epoch = jnp.full((1,), cur_epoch, jnp.int32)
    out = _sel2(h2, a2, st, psums, epoch)
    return out.reshape(())
